# baseline (speedup 1.0000x reference)
import functools

import jax
import jax.numpy as jnp
from jax import lax
from jax.experimental import pallas as pl
from jax.experimental.pallas import tpu as pltpu
from jax.experimental.pallas import tpu_sc as plsc

D = 1024
L = 16
VPR = D // L

_info = plsc.get_sparse_core_info()
NC, NS = _info.num_cores, _info.num_subcores
NW = NC * NS

SC_BATCHES = 2
TC_BLK = 512


def _sc_part(x2, encoding, S):
    R = x2.shape[0]
    rows_per_w = R // NW
    C = 32
    NB = 3
    n_chunks = rows_per_w // C

    mesh = plsc.VectorSubcoreMesh(core_axis_name="c", subcore_axis_name="s")

    @functools.partial(
        pl.kernel,
        out_type=jax.ShapeDtypeStruct((R, D), jnp.float32),
        mesh=mesh,
        scratch_types=(
            [pltpu.VMEM((2, D), jnp.float32)]
            + [pltpu.VMEM((C, D), jnp.float32)] * NB
            + [pltpu.SemaphoreType.DMA] * (2 * NB)
        ),
    )
    def k(x_hbm, enc_hbm, out_hbm, enc_v, *rest):
        bufs = rest[:NB]
        isems = rest[NB:2 * NB]
        osems = rest[2 * NB:3 * NB]

        wid = lax.axis_index("s") * NC + lax.axis_index("c")
        wstart = wid * rows_per_w
        pltpu.sync_copy(enc_hbm.at[pl.ds(0, 2)], enc_v)

        def start_in(i, b):
            pltpu.async_copy(x_hbm.at[pl.ds(wstart + i * C, C)], bufs[b],
                             isems[b])

        def wait_in(b):
            pltpu.make_async_copy(x_hbm.at[pl.ds(0, C)], bufs[b],
                                  isems[b]).wait()

        def start_out(i, b):
            pltpu.async_copy(bufs[b], out_hbm.at[pl.ds(wstart + i * C, C)],
                             osems[b])

        def wait_out(b):
            pltpu.make_async_copy(bufs[b], out_hbm.at[pl.ds(0, C)],
                                  osems[b]).wait()

        def add_rows(buf):
            H = VPR // 2
            for half in range(2):
                evs = [enc_v[0, pl.ds((half * H + j) * L, L)] for j in range(H)]

                def row_body(r):
                    for j in range(H):
                        buf[r, pl.ds((half * H + j) * L, L)] += evs[j]

                plsc.parallel_loop(0, C, 1, unroll=2)(row_body)

        start_in(0, 0)
        start_in(1, 1)

        def step(i, b):
            b2 = (b + 2) % NB

            @pl.when((i + 2 < n_chunks) & (i >= 1))
            def _drain():
                wait_out(b2)

            @pl.when(i + 2 < n_chunks)
            def _prefetch():
                start_in(i + 2, b2)

            wait_in(b)
            add_rows(bufs[b])
            start_out(i, b)

        def outer(h, carry):
            for b in range(NB):
                step(NB * h + b, b)
            return carry

        n_full = (n_chunks // NB) * NB
        lax.fori_loop(0, n_chunks // NB, outer, 0)
        for i in range(n_full, n_chunks):
            step(i, i % NB)
        for i in range(n_chunks - 3, n_chunks):
            wait_out(i % NB)

        last = wstart + rows_per_w - 1

        @pl.when((last + 1) % S == 0)
        def _fix():
            pltpu.sync_copy(x_hbm.at[pl.ds(last - 1, 1)],
                            bufs[0].at[pl.ds(0, 1)])
            for j in range(VPR):
                sl = pl.ds(j * L, L)
                bufs[0][0, sl] += enc_v[1, sl]
            pltpu.sync_copy(bufs[0].at[pl.ds(0, 1)],
                            out_hbm.at[pl.ds(last, 1)])

    return k(x2, encoding)


def _tc_part(x2, encoding, S):
    R = x2.shape[0]
    grid = R // TC_BLK

    def body(x_ref, enc_ref, o_ref):
        o_ref[...] = x_ref[...] + enc_ref[0:1, :]
        blk_end = (pl.program_id(0) + 1) * TC_BLK

        @pl.when(blk_end % S == 0)
        def _fix():
            o_ref[TC_BLK - 1:TC_BLK, :] = (
                x_ref[TC_BLK - 2:TC_BLK - 1, :] + enc_ref[1:2, :])

    return pl.pallas_call(
        body,
        grid=(grid,),
        in_specs=[
            pl.BlockSpec((TC_BLK, D), lambda i: (i, 0)),
            pl.BlockSpec((2, D), lambda i: (0, 0)),
        ],
        out_specs=pl.BlockSpec((TC_BLK, D), lambda i: (i, 0)),
        out_shape=jax.ShapeDtypeStruct((R, D), jnp.float32),
    )(x2, encoding[:2])


def kernel(x, encoding):
    B, S, d = x.shape
    assert d == D
    bs = SC_BATCHES
    out_sc = _sc_part(x[:bs].reshape(bs * S, D), encoding, S)
    out_tc = _tc_part(x[bs:].reshape((B - bs) * S, D), encoding, S)
    return jnp.concatenate(
        [out_sc.reshape(bs, S, D), out_tc.reshape(B - bs, S, D)], axis=0)


# speedup vs baseline: 3.1491x; 3.1491x over previous
import functools

import jax
import jax.numpy as jnp
from jax import lax
from jax.experimental import pallas as pl
from jax.experimental.pallas import tpu as pltpu
from jax.experimental.pallas import tpu_sc as plsc

D = 1024
L = 16
VPR = D // L

_info = plsc.get_sparse_core_info()
NC, NS = _info.num_cores, _info.num_subcores
NW = NC * NS

SC_BATCHES = 2
TC_BLK = 512


def _sc_part(x2, encoding, S):
    R = x2.shape[0]
    rows_per_w = R // NW
    C = 32
    NB = 3
    n_chunks = rows_per_w // C

    mesh = plsc.VectorSubcoreMesh(core_axis_name="c", subcore_axis_name="s")

    @functools.partial(
        pl.kernel,
        out_type=jax.ShapeDtypeStruct((R, D), jnp.float32),
        mesh=mesh,
        scratch_types=(
            [pltpu.VMEM((2, D), jnp.float32)]
            + [pltpu.VMEM((C, D), jnp.float32)] * NB
            + [pltpu.SemaphoreType.DMA] * (2 * NB)
        ),
    )
    def k(x_hbm, enc_hbm, out_hbm, enc_v, *rest):
        bufs = rest[:NB]
        isems = rest[NB:2 * NB]
        osems = rest[2 * NB:3 * NB]

        wid = lax.axis_index("s") * NC + lax.axis_index("c")
        wstart = wid * rows_per_w
        pltpu.sync_copy(enc_hbm.at[pl.ds(0, 2)], enc_v)

        def start_in(i, b):
            pltpu.async_copy(x_hbm.at[pl.ds(wstart + i * C, C)], bufs[b],
                             isems[b])

        def wait_in(b):
            pltpu.make_async_copy(x_hbm.at[pl.ds(0, C)], bufs[b],
                                  isems[b]).wait()

        def start_out(i, b):
            pltpu.async_copy(bufs[b], out_hbm.at[pl.ds(wstart + i * C, C)],
                             osems[b])

        def wait_out(b):
            pltpu.make_async_copy(bufs[b], out_hbm.at[pl.ds(0, C)],
                                  osems[b]).wait()

        def add_rows(buf):
            H = VPR // 2
            for half in range(2):
                evs = [enc_v[0, pl.ds((half * H + j) * L, L)] for j in range(H)]

                def row_body(r):
                    for j in range(H):
                        buf[r, pl.ds((half * H + j) * L, L)] += evs[j]

                plsc.parallel_loop(0, C, 1, unroll=2)(row_body)

        start_in(0, 0)
        start_in(1, 1)

        def step(i, b):
            b2 = (b + 2) % NB

            @pl.when((i + 2 < n_chunks) & (i >= 1))
            def _drain():
                wait_out(b2)

            @pl.when(i + 2 < n_chunks)
            def _prefetch():
                start_in(i + 2, b2)

            wait_in(b)
            add_rows(bufs[b])
            start_out(i, b)

        def outer(h, carry):
            for b in range(NB):
                step(NB * h + b, b)
            return carry

        n_full = (n_chunks // NB) * NB
        lax.fori_loop(0, n_chunks // NB, outer, 0)
        for i in range(n_full, n_chunks):
            step(i, i % NB)
        for i in range(n_chunks - 3, n_chunks):
            wait_out(i % NB)

        last = wstart + rows_per_w - 1

        @pl.when((last + 1) % S == 0)
        def _fix():
            pltpu.sync_copy(x_hbm.at[pl.ds(last - 1, 1)],
                            bufs[0].at[pl.ds(0, 1)])
            for j in range(VPR):
                sl = pl.ds(j * L, L)
                bufs[0][0, sl] += enc_v[1, sl]
            pltpu.sync_copy(bufs[0].at[pl.ds(0, 1)],
                            out_hbm.at[pl.ds(last, 1)])

    return k(x2, encoding)


def _tc_part(x2, encoding, S):
    R = x2.shape[0]
    grid = R // TC_BLK

    def body(x_ref, enc_ref, o_ref):
        o_ref[...] = x_ref[...] + enc_ref[0:1, :]
        blk_end = (pl.program_id(0) + 1) * TC_BLK

        @pl.when(blk_end % S == 0)
        def _fix():
            o_ref[TC_BLK - 1:TC_BLK, :] = (
                x_ref[TC_BLK - 2:TC_BLK - 1, :] + enc_ref[1:2, :])

    return pl.pallas_call(
        body,
        grid=(grid,),
        in_specs=[
            pl.BlockSpec((TC_BLK, D), lambda i: (i, 0)),
            pl.BlockSpec((2, D), lambda i: (0, 0)),
        ],
        out_specs=pl.BlockSpec((TC_BLK, D), lambda i: (i, 0)),
        out_shape=jax.ShapeDtypeStruct((R, D), jnp.float32),
    )(x2, encoding[:2])


def kernel(x, encoding):
    B, S, d = x.shape
    assert d == D
    out_tc = _tc_part(x.reshape(B * S, D), encoding, S)
    return out_tc.reshape(B, S, D)
